# baseline (speedup 1.0000x reference)
import functools

import jax
import jax.numpy as jnp
from jax import lax
from jax.experimental import pallas as pl
from jax.experimental.pallas import tpu as pltpu
from jax.experimental.pallas import tpu_sc as plsc

_REP = 64


def kernel(x, pos_emb):
    B, T = x.shape
    D = pos_emb.shape[1]
    TD = T * D
    pe = pos_emb[:T].reshape(1, TD)

    info = plsc.get_sparse_core_info()
    nc = info.num_cores
    rows_per_core = B // nc
    n_copies = rows_per_core // _REP

    mesh = plsc.ScalarSubcoreMesh(axis_name="c", num_cores=nc)

    @functools.partial(
        pl.kernel,
        mesh=mesh,
        out_type=jax.ShapeDtypeStruct((B, TD), jnp.float32),
        scratch_types=[
            pltpu.VMEM_SHARED((_REP, TD), jnp.float32),
            pltpu.SemaphoreType.DMA,
            pltpu.SemaphoreType.DMA,
        ],
    )
    def sc_broadcast(pe_hbm, out_hbm, shared, lsem, ssem):
        cid = lax.axis_index("c")
        base = cid * rows_per_core
        for i in range(_REP):
            pltpu.async_copy(pe_hbm, shared.at[pl.ds(i, 1)], lsem)
        for i in range(_REP):
            pltpu.make_async_copy(pe_hbm, shared.at[pl.ds(i, 1)], lsem).wait()
        for j in range(n_copies):
            pltpu.async_copy(shared, out_hbm.at[pl.ds(base + j * _REP, _REP)], ssem)
        for j in range(n_copies):
            pltpu.make_async_copy(
                shared, out_hbm.at[pl.ds(base + j * _REP, _REP)], ssem
            ).wait()

    out = sc_broadcast(pe)
    return out.reshape(B, T, D)


# speedup vs baseline: 1.0708x; 1.0708x over previous
import jax
import jax.numpy as jnp
from jax import lax
from jax._src.pallas import mpmd
from jax.experimental import pallas as pl
from jax.experimental.pallas import tpu as pltpu
from jax.experimental.pallas import tpu_sc as plsc

_K = 4
_REP = 64
_B_VEC = 2048


def kernel(x, pos_emb):
    B, T = x.shape
    D = pos_emb.shape[1]
    TD = T * D
    pe = pos_emb[:T].reshape(1, TD)

    info = plsc.get_sparse_core_info()
    nc, ns = info.num_cores, info.num_subcores
    nw = nc * ns
    rows_v = _B_VEC // nw
    nv_copies = rows_v // _K
    b_scs = B - _B_VEC
    rows_s = b_scs // nc
    ns_copies = rows_s // _REP

    vec_mesh = plsc.VectorSubcoreMesh(core_axis_name="c", subcore_axis_name="s")
    scs_mesh = plsc.ScalarSubcoreMesh(axis_name="c", num_cores=nc)

    def vec_fn(pe_hbm, out_hbm, shared):
        del shared

        def inner(buf, lsem, ssem):
            wid = lax.axis_index("s") * nc + lax.axis_index("c")
            base = wid * rows_v
            for i in range(_K):
                pltpu.async_copy(pe_hbm, buf.at[pl.ds(i, 1)], lsem)
            for i in range(_K):
                pltpu.make_async_copy(pe_hbm, buf.at[pl.ds(i, 1)], lsem).wait()
            for j in range(nv_copies):
                pltpu.async_copy(buf, out_hbm.at[pl.ds(base + j * _K, _K)], ssem)
            for j in range(nv_copies):
                pltpu.make_async_copy(
                    buf, out_hbm.at[pl.ds(base + j * _K, _K)], ssem
                ).wait()

        pl.run_scoped(
            inner,
            pltpu.VMEM((_K, TD), jnp.float32),
            pltpu.SemaphoreType.DMA,
            pltpu.SemaphoreType.DMA,
        )

    def scs_fn(pe_hbm, out_hbm, shared):
        def inner(lsem, ssem):
            cid = lax.axis_index("c")
            base = _B_VEC + cid * rows_s
            for i in range(_REP):
                pltpu.async_copy(pe_hbm, shared.at[pl.ds(i, 1)], lsem)
            for i in range(_REP):
                pltpu.make_async_copy(pe_hbm, shared.at[pl.ds(i, 1)], lsem).wait()
            for j in range(ns_copies):
                pltpu.async_copy(
                    shared, out_hbm.at[pl.ds(base + j * _REP, _REP)], ssem
                )
            for j in range(ns_copies):
                pltpu.make_async_copy(
                    shared, out_hbm.at[pl.ds(base + j * _REP, _REP)], ssem
                ).wait()

        pl.run_scoped(
            inner, pltpu.SemaphoreType.DMA, pltpu.SemaphoreType.DMA
        )

    out = mpmd.mpmd_map(
        [(vec_mesh, vec_fn), (scs_mesh, scs_fn)],
        out_types=jax.ShapeDtypeStruct((B, TD), jnp.float32),
        scratch_types=[pltpu.VMEM_SHARED((_REP, TD), jnp.float32)],
    )(pe)
    return out.reshape(B, T, D)


# speedup vs baseline: 1.1224x; 1.0482x over previous
import functools

import jax
import jax.numpy as jnp
from jax import lax
from jax.experimental import pallas as pl
from jax.experimental.pallas import tpu as pltpu
from jax.experimental.pallas import tpu_sc as plsc

_K = 4


def kernel(x, pos_emb):
    B, T = x.shape
    D = pos_emb.shape[1]
    TD = T * D
    pe = pos_emb[:T].reshape(1, TD)

    info = plsc.get_sparse_core_info()
    nw = info.num_cores * info.num_subcores
    rows_per_w = B // nw
    n_copies = rows_per_w // _K

    mesh = plsc.VectorSubcoreMesh(core_axis_name="c", subcore_axis_name="s")

    @functools.partial(
        pl.kernel,
        mesh=mesh,
        out_type=jax.ShapeDtypeStruct((B, TD), jnp.float32),
        scratch_types=[
            pltpu.VMEM((_K, TD), jnp.float32),
            pltpu.SemaphoreType.DMA,
            pltpu.SemaphoreType.DMA,
        ],
    )
    def sc_broadcast(pe_hbm, out_hbm, buf, lsem, ssem):
        wid = lax.axis_index("s") * info.num_cores + lax.axis_index("c")
        base = wid * rows_per_w
        for i in range(_K):
            pltpu.async_copy(pe_hbm, buf.at[pl.ds(i, 1)], lsem)
        for i in range(_K):
            pltpu.make_async_copy(pe_hbm, buf.at[pl.ds(i, 1)], lsem).wait()
        for j in range(n_copies):
            pltpu.async_copy(buf, out_hbm.at[pl.ds(base + j * _K, _K)], ssem)
        for j in range(n_copies):
            pltpu.make_async_copy(
                buf, out_hbm.at[pl.ds(base + j * _K, _K)], ssem
            ).wait()

    out = sc_broadcast(pe)
    return out.reshape(B, T, D)


# speedup vs baseline: 1.1394x; 1.0152x over previous
import functools

import jax
import jax.numpy as jnp
from jax import lax
from jax.experimental import pallas as pl
from jax.experimental.pallas import tpu as pltpu
from jax.experimental.pallas import tpu_sc as plsc

_K = 2


def kernel(x, pos_emb):
    B, T = x.shape
    D = pos_emb.shape[1]
    TD = T * D
    pe = pos_emb[:T].reshape(1, TD)

    info = plsc.get_sparse_core_info()
    nw = info.num_cores * info.num_subcores
    rows_per_w = B // nw
    n_copies = rows_per_w // _K

    mesh = plsc.VectorSubcoreMesh(core_axis_name="c", subcore_axis_name="s")

    @functools.partial(
        pl.kernel,
        mesh=mesh,
        out_type=jax.ShapeDtypeStruct((B, TD), jnp.float32),
        scratch_types=[
            pltpu.VMEM((_K, TD), jnp.float32),
            pltpu.SemaphoreType.DMA,
            pltpu.SemaphoreType.DMA,
        ],
    )
    def sc_broadcast(pe_hbm, out_hbm, buf, lsem, ssem):
        wid = lax.axis_index("s") * info.num_cores + lax.axis_index("c")
        base = wid * rows_per_w
        for i in range(_K):
            pltpu.async_copy(pe_hbm, buf.at[pl.ds(i, 1)], lsem)
        for i in range(_K):
            pltpu.make_async_copy(pe_hbm, buf.at[pl.ds(i, 1)], lsem).wait()
        for j in range(n_copies):
            pltpu.async_copy(buf, out_hbm.at[pl.ds(base + j * _K, _K)], ssem)
        for j in range(n_copies):
            pltpu.make_async_copy(
                buf, out_hbm.at[pl.ds(base + j * _K, _K)], ssem
            ).wait()

    out = sc_broadcast(pe)
    return out.reshape(B, T, D)


# speedup vs baseline: 1.1440x; 1.0040x over previous
import functools

import jax
import jax.numpy as jnp
from jax import lax
from jax.experimental import pallas as pl
from jax.experimental.pallas import tpu as pltpu
from jax.experimental.pallas import tpu_sc as plsc

_K = 1


def kernel(x, pos_emb):
    B, T = x.shape
    D = pos_emb.shape[1]
    TD = T * D
    pe = pos_emb[:T].reshape(1, TD)

    info = plsc.get_sparse_core_info()
    nw = info.num_cores * info.num_subcores
    rows_per_w = B // nw
    n_copies = rows_per_w // _K

    mesh = plsc.VectorSubcoreMesh(core_axis_name="c", subcore_axis_name="s")

    @functools.partial(
        pl.kernel,
        mesh=mesh,
        out_type=jax.ShapeDtypeStruct((B, TD), jnp.float32),
        scratch_types=[
            pltpu.VMEM((_K, TD), jnp.float32),
            pltpu.SemaphoreType.DMA,
            pltpu.SemaphoreType.DMA,
        ],
    )
    def sc_broadcast(pe_hbm, out_hbm, buf, lsem, ssem):
        wid = lax.axis_index("s") * info.num_cores + lax.axis_index("c")
        base = wid * rows_per_w
        for i in range(_K):
            pltpu.async_copy(pe_hbm, buf.at[pl.ds(i, 1)], lsem)
        for i in range(_K):
            pltpu.make_async_copy(pe_hbm, buf.at[pl.ds(i, 1)], lsem).wait()
        for j in range(n_copies):
            pltpu.async_copy(buf, out_hbm.at[pl.ds(base + j * _K, _K)], ssem)
        for j in range(n_copies):
            pltpu.make_async_copy(
                buf, out_hbm.at[pl.ds(base + j * _K, _K)], ssem
            ).wait()

    out = sc_broadcast(pe)
    return out.reshape(B, T, D)
